# baseline (speedup 1.0000x reference)
import functools

import jax
import jax.numpy as jnp
from jax import lax
from jax.experimental import pallas as pl
from jax.experimental.pallas import tpu as pltpu
from jax.experimental.pallas import tpu_sc as plsc

N_NODES = 10000
N_EDGES = 320000
DEGREE = 32
NODE_SIZE = 128
EDGE_SIZE = 16
OUT_SIZE = 128
EPS = 1e-5

NUM_WORKERS = 32
LANES = 16
IDX_ROW = 128
IDX_ROWS_TOTAL = N_NODES * DEGREE // IDX_ROW
IDX_PAD_ROWS = 2504
NBUF = 2

CHUNK_N = 8
MAX_CPW_N = -(-(N_NODES // CHUNK_N) // NUM_WORKERS)
CHUNK_E = 16
MAX_CPW_E = -(-(N_NODES // CHUNK_E) // NUM_WORKERS)

RP_CHUNK = 256
RP_TOTAL = N_EDGES // RP_CHUNK
MAX_RP = -(-RP_TOTAL // NUM_WORKERS)
PACKED_ROWS = N_EDGES * EDGE_SIZE // 128


def _worker_id():
    return lax.axis_index("s") * 2 + lax.axis_index("c")


def _repack_body(et_hbm, out_hbm, in_bufs, out_bufs, sems):
    wid = _worker_id()
    base = wid * RP_TOTAL // NUM_WORKERS
    cnt = (wid + 1) * RP_TOTAL // NUM_WORKERS - base

    def issue(k, b):
        pltpu.async_copy(et_hbm.at[:, pl.ds((base + k) * RP_CHUNK, RP_CHUNK)],
                         in_bufs[b], sems[b])

    def drain(k, b):
        pltpu.make_async_copy(
            et_hbm.at[:, pl.ds((base + k) * RP_CHUNK, RP_CHUNK)],
            in_bufs[b], sems[b]).wait()

    feat = lax.iota(jnp.int32, LANES)

    def process(k, b):
        def row_body(r, carry):
            for sub in range(8):
                e = r * 8 + sub
                col = jnp.full((LANES,), e, jnp.int32)
                vals = plsc.load_gather(in_bufs[b], [feat, col])
                out_bufs[b][r, pl.ds(sub * LANES, LANES)] = vals
            return carry

        lax.fori_loop(0, RP_CHUNK // 8, row_body, 0)
        pltpu.sync_copy(out_bufs[b],
                        out_hbm.at[pl.ds((base + k) * (RP_CHUNK // 8),
                                         RP_CHUNK // 8)])

    for b in range(NBUF):
        issue(b, b)

    def pair_body(i, carry):
        for b in range(NBUF):
            k = i * NBUF + b

            @pl.when(k < cnt)
            def _():
                drain(k, b)
                process(k, b)

                @pl.when(k + NBUF < cnt)
                def _():
                    issue(k + NBUF, b)

        return carry

    lax.fori_loop(0, -(-MAX_RP // NBUF), pair_body, 0)


def _repack_edges(edge_t):
    mesh = plsc.VectorSubcoreMesh(core_axis_name="c", subcore_axis_name="s")
    kern = pl.kernel(
        _repack_body,
        mesh=mesh,
        compiler_params=pltpu.CompilerParams(use_tc_tiling_on_sc=True,
                                             needs_layout_passes=False),
        out_type=jax.ShapeDtypeStruct((PACKED_ROWS, 128), jnp.float32),
        scratch_types=[
            [pltpu.VMEM((EDGE_SIZE, RP_CHUNK), jnp.float32)
             for _ in range(NBUF)],
            [pltpu.VMEM((RP_CHUNK // 8, 128), jnp.float32)
             for _ in range(NBUF)],
            [pltpu.SemaphoreType.DMA for _ in range(NBUF)],
        ],
    )
    return kern(edge_t)


def _gather_sum_body(table_hbm, idx_hbm, out_hbm, idx_v, rows, out_v, sems,
                     *, width, chunk, max_cpw, aligned):
    nv = width // LANES
    rows_per_chunk = chunk * DEGREE // IDX_ROW
    wid = _worker_id()
    base = wid * (N_NODES // chunk) // NUM_WORKERS
    cnt = (wid + 1) * (N_NODES // chunk) // NUM_WORKERS - base
    irow0 = base * rows_per_chunk
    n_irows = max_cpw * rows_per_chunk

    if aligned:
        al = (irow0 // 8) * 8
        off = irow0 - al
        pltpu.sync_copy(idx_hbm.at[pl.ds(al, n_irows + 8)], idx_v)
    else:
        off = 0
        pltpu.sync_copy(idx_hbm.at[pl.ds(irow0, n_irows)], idx_v)

    def issue(k, b):
        for j in range(rows_per_chunk):
            pltpu.async_copy(
                table_hbm.at[idx_v.at[k * rows_per_chunk + j + off]],
                rows[b].at[pl.ds(j * IDX_ROW, IDX_ROW)], sems[b])

    def drain(k, b):
        for j in range(rows_per_chunk):
            pltpu.make_async_copy(
                table_hbm.at[idx_v.at[k * rows_per_chunk + j + off]],
                rows[b].at[pl.ds(j * IDX_ROW, IDX_ROW)], sems[b]).wait()

    def reduce_chunk(k, b):
        for n in range(chunk):
            def red(j, acc):
                new = acc
                for jj in range(4):
                    row = n * DEGREE + j * 4 + jj
                    new = tuple(
                        new[v] + rows[b][row, pl.ds(v * LANES, LANES)]
                        for v in range(nv)
                    )
                return new

            zero = jnp.zeros((LANES,), jnp.float32)
            acc = lax.fori_loop(0, DEGREE // 4, red, (zero,) * nv)
            out_row = k * chunk + n
            for v in range(nv):
                out_v[out_row, pl.ds(v * LANES, LANES)] = acc[v]

    for b in range(NBUF):
        issue(b, b)

    def pair_body(i, carry):
        for b in range(NBUF):
            k = i * NBUF + b

            @pl.when(k < cnt)
            def _():
                drain(k, b)
                reduce_chunk(k, b)

                @pl.when(k + NBUF < cnt)
                def _():
                    issue(k + NBUF, b)

        return carry

    lax.fori_loop(0, -(-max_cpw // NBUF), pair_body, 0)

    row0 = base * chunk

    @pl.when(cnt == max_cpw)
    def _():
        pltpu.sync_copy(out_v, out_hbm.at[pl.ds(row0, max_cpw * chunk)])

    @pl.when(cnt == max_cpw - 1)
    def _():
        small = (max_cpw - 1) * chunk
        pltpu.sync_copy(out_v.at[pl.ds(0, small)],
                        out_hbm.at[pl.ds(row0, small)])


def _make_sc_kernel(width, chunk, max_cpw, aligned, n_dummy=0):
    def body(*refs):
        _gather_sum_body(*refs[:2], *refs[2 + n_dummy:], width=width,
                         chunk=chunk, max_cpw=max_cpw, aligned=aligned)

    mesh = plsc.VectorSubcoreMesh(core_axis_name="c", subcore_axis_name="s")
    rows_per_chunk = chunk * DEGREE // IDX_ROW
    idx_rows = max_cpw * rows_per_chunk + (8 if aligned else 0)
    return pl.kernel(
        body,
        mesh=mesh,
        compiler_params=pltpu.CompilerParams(use_tc_tiling_on_sc=aligned),
        out_type=jax.ShapeDtypeStruct((N_NODES, width), jnp.float32),
        scratch_types=[
            pltpu.VMEM((idx_rows, IDX_ROW), jnp.int32),
            [pltpu.VMEM((rows_per_chunk * IDX_ROW, width), jnp.float32)
             for _ in range(NBUF)],
            pltpu.VMEM((max_cpw * chunk, width), jnp.float32),
            [pltpu.SemaphoreType.DMA for _ in range(NBUF)],
        ],
    )


@jax.jit
def _sc_gather_sums(node_repr, edge_t, nn2d, en2d):
    edge_packed = _repack_edges(edge_t)
    edge_lin = edge_packed.reshape(N_EDGES, EDGE_SIZE)
    nsum = _make_sc_kernel(NODE_SIZE, CHUNK_N, MAX_CPW_N, True)(
        node_repr, nn2d)
    esum = _make_sc_kernel(EDGE_SIZE, CHUNK_E, MAX_CPW_E, False, n_dummy=1)(
        edge_lin, en2d, nsum)
    return nsum, esum


def _tc_body(nsum_ref, esum_ref, node_ref, wdn_ref, wde_ref, ws_ref, bias_ref,
             out_ref):
    act = jnp.dot(nsum_ref[:], wdn_ref[:], preferred_element_type=jnp.float32)
    act = act + jnp.dot(esum_ref[:], wde_ref[:],
                        preferred_element_type=jnp.float32)
    act = act + jnp.dot(node_ref[:], ws_ref[:],
                        preferred_element_type=jnp.float32)
    act = act + bias_ref[:]
    mean = jnp.mean(act, axis=0, keepdims=True)
    cent = act - mean
    var = jnp.mean(cent * cent, axis=0, keepdims=True)
    out_ref[:] = jnp.maximum(cent * lax.rsqrt(var + EPS), 0.0)


def _tc_finish(nsum, esum, node_repr, wdn, wde, ws, bias):
    return pl.pallas_call(
        _tc_body,
        out_shape=jax.ShapeDtypeStruct((N_NODES, OUT_SIZE), jnp.float32),
    )(nsum, esum, node_repr, wdn, wde, ws, bias)


def kernel(node_repr, edge_repr, node_neighbor, edge_neighbor, W_deg, W_self,
           bias):
    nn2d = jnp.pad(node_neighbor.reshape(IDX_ROWS_TOTAL, IDX_ROW),
                   ((0, IDX_PAD_ROWS - IDX_ROWS_TOTAL), (0, 0)))
    en2d = edge_neighbor.reshape(IDX_ROWS_TOTAL, IDX_ROW)
    nsum, esum = _sc_gather_sums(node_repr, edge_repr.T, nn2d, en2d)
    return _tc_finish(nsum, esum, node_repr, W_deg[:NODE_SIZE],
                      W_deg[NODE_SIZE:], W_self, bias)


# speedup vs baseline: 1.3986x; 1.3986x over previous
import functools

import jax
import jax.numpy as jnp
from jax import lax
from jax.experimental import pallas as pl
from jax.experimental.pallas import tpu as pltpu
from jax.experimental.pallas import tpu_sc as plsc

N_NODES = 10000
N_EDGES = 320000
DEGREE = 32
NODE_SIZE = 128
EDGE_SIZE = 16
OUT_SIZE = 128
EPS = 1e-5

NUM_WORKERS = 32
LANES = 16
IDX_ROW = 128
IDX_ROWS_TOTAL = N_NODES * DEGREE // IDX_ROW
IDX_PAD_ROWS = 2504
NBUF = 2

CHUNK_N = 8
MAX_CPW_N = -(-(N_NODES // CHUNK_N) // NUM_WORKERS)
CHUNK_E = 16
MAX_CPW_E = -(-(N_NODES // CHUNK_E) // NUM_WORKERS)

RP_BLK = 3200
PACKED_ROWS = N_EDGES * EDGE_SIZE // 128


def _worker_id():
    return lax.axis_index("s") * 2 + lax.axis_index("c")


def _tc_repack_body(in_ref, sel_ref, out_ref):
    x = in_ref[:]
    eye = jnp.eye(EDGE_SIZE, dtype=jnp.float32)
    t = lax.dot_general(x, eye, (((0,), (0,)), ((), ())),
                        preferred_element_type=jnp.float32)
    t3 = t.reshape(RP_BLK // 8, 8, EDGE_SIZE)
    acc = jnp.dot(t3[:, 0, :], sel_ref[0],
                  preferred_element_type=jnp.float32)
    for s in range(1, 8):
        acc = acc + jnp.dot(t3[:, s, :], sel_ref[s],
                            preferred_element_type=jnp.float32)
    out_ref[:] = acc


def _repack_edges(edge_t):
    sel = jnp.eye(128, dtype=jnp.float32).reshape(8, EDGE_SIZE, 128)
    return pl.pallas_call(
        _tc_repack_body,
        grid=(N_EDGES // RP_BLK,),
        in_specs=[pl.BlockSpec((EDGE_SIZE, RP_BLK), lambda i: (0, i)),
                  pl.BlockSpec((8, EDGE_SIZE, 128), lambda i: (0, 0, 0))],
        out_specs=pl.BlockSpec((RP_BLK // 8, 128), lambda i: (i, 0)),
        out_shape=jax.ShapeDtypeStruct((PACKED_ROWS, 128), jnp.float32),
    )(edge_t, sel)


def _gather_sum_body(table_hbm, idx_hbm, out_hbm, idx_v, rows, out_v, sems,
                     *, width, chunk, max_cpw, aligned):
    nv = width // LANES
    rows_per_chunk = chunk * DEGREE // IDX_ROW
    wid = _worker_id()
    base = wid * (N_NODES // chunk) // NUM_WORKERS
    cnt = (wid + 1) * (N_NODES // chunk) // NUM_WORKERS - base
    irow0 = base * rows_per_chunk
    n_irows = max_cpw * rows_per_chunk

    if aligned:
        al = (irow0 // 8) * 8
        off = irow0 - al
        pltpu.sync_copy(idx_hbm.at[pl.ds(al, n_irows + 8)], idx_v)
    else:
        off = 0
        pltpu.sync_copy(idx_hbm.at[pl.ds(irow0, n_irows)], idx_v)

    def issue(k, b):
        for j in range(rows_per_chunk):
            pltpu.async_copy(
                table_hbm.at[idx_v.at[k * rows_per_chunk + j + off]],
                rows[b].at[pl.ds(j * IDX_ROW, IDX_ROW)], sems[b])

    def drain(k, b):
        for j in range(rows_per_chunk):
            pltpu.make_async_copy(
                table_hbm.at[idx_v.at[k * rows_per_chunk + j + off]],
                rows[b].at[pl.ds(j * IDX_ROW, IDX_ROW)], sems[b]).wait()

    def reduce_chunk(k, b):
        for n in range(chunk):
            def red(j, acc):
                new = acc
                for jj in range(4):
                    row = n * DEGREE + j * 4 + jj
                    new = tuple(
                        new[v] + rows[b][row, pl.ds(v * LANES, LANES)]
                        for v in range(nv)
                    )
                return new

            zero = jnp.zeros((LANES,), jnp.float32)
            acc = lax.fori_loop(0, DEGREE // 4, red, (zero,) * nv)
            out_row = k * chunk + n
            for v in range(nv):
                out_v[out_row, pl.ds(v * LANES, LANES)] = acc[v]

    for b in range(NBUF):
        issue(b, b)

    def pair_body(i, carry):
        for b in range(NBUF):
            k = i * NBUF + b

            @pl.when(k < cnt)
            def _():
                drain(k, b)
                reduce_chunk(k, b)

                @pl.when(k + NBUF < cnt)
                def _():
                    issue(k + NBUF, b)

        return carry

    lax.fori_loop(0, -(-max_cpw // NBUF), pair_body, 0)

    row0 = base * chunk

    @pl.when(cnt == max_cpw)
    def _():
        pltpu.sync_copy(out_v, out_hbm.at[pl.ds(row0, max_cpw * chunk)])

    @pl.when(cnt == max_cpw - 1)
    def _():
        small = (max_cpw - 1) * chunk
        pltpu.sync_copy(out_v.at[pl.ds(0, small)],
                        out_hbm.at[pl.ds(row0, small)])


def _make_sc_kernel(width, chunk, max_cpw, aligned, n_dummy=0):
    def body(*refs):
        _gather_sum_body(*refs[:2], *refs[2 + n_dummy:], width=width,
                         chunk=chunk, max_cpw=max_cpw, aligned=aligned)

    mesh = plsc.VectorSubcoreMesh(core_axis_name="c", subcore_axis_name="s")
    rows_per_chunk = chunk * DEGREE // IDX_ROW
    idx_rows = max_cpw * rows_per_chunk + (8 if aligned else 0)
    return pl.kernel(
        body,
        mesh=mesh,
        compiler_params=pltpu.CompilerParams(use_tc_tiling_on_sc=aligned),
        out_type=jax.ShapeDtypeStruct((N_NODES, width), jnp.float32),
        scratch_types=[
            pltpu.VMEM((idx_rows, IDX_ROW), jnp.int32),
            [pltpu.VMEM((rows_per_chunk * IDX_ROW, width), jnp.float32)
             for _ in range(NBUF)],
            pltpu.VMEM((max_cpw * chunk, width), jnp.float32),
            [pltpu.SemaphoreType.DMA for _ in range(NBUF)],
        ],
    )


@jax.jit
def _sc_gather_sums(node_repr, edge_t, nn2d, en2d):
    edge_packed = _repack_edges(edge_t)
    edge_lin = edge_packed.reshape(N_EDGES, EDGE_SIZE)
    nsum = _make_sc_kernel(NODE_SIZE, CHUNK_N, MAX_CPW_N, True)(
        node_repr, nn2d)
    esum = _make_sc_kernel(EDGE_SIZE, CHUNK_E, MAX_CPW_E, False, n_dummy=1)(
        edge_lin, en2d, nsum)
    return nsum, esum


def _tc_body(nsum_ref, esum_ref, node_ref, wdn_ref, wde_ref, ws_ref, bias_ref,
             out_ref):
    act = jnp.dot(nsum_ref[:], wdn_ref[:], preferred_element_type=jnp.float32)
    act = act + jnp.dot(esum_ref[:], wde_ref[:],
                        preferred_element_type=jnp.float32)
    act = act + jnp.dot(node_ref[:], ws_ref[:],
                        preferred_element_type=jnp.float32)
    act = act + bias_ref[:]
    mean = jnp.mean(act, axis=0, keepdims=True)
    cent = act - mean
    var = jnp.mean(cent * cent, axis=0, keepdims=True)
    out_ref[:] = jnp.maximum(cent * lax.rsqrt(var + EPS), 0.0)


def _tc_finish(nsum, esum, node_repr, wdn, wde, ws, bias):
    return pl.pallas_call(
        _tc_body,
        out_shape=jax.ShapeDtypeStruct((N_NODES, OUT_SIZE), jnp.float32),
    )(nsum, esum, node_repr, wdn, wde, ws, bias)


def kernel(node_repr, edge_repr, node_neighbor, edge_neighbor, W_deg, W_self,
           bias):
    nn2d = jnp.pad(node_neighbor.reshape(IDX_ROWS_TOTAL, IDX_ROW),
                   ((0, IDX_PAD_ROWS - IDX_ROWS_TOTAL), (0, 0)))
    en2d = edge_neighbor.reshape(IDX_ROWS_TOTAL, IDX_ROW)
    nsum, esum = _sc_gather_sums(node_repr, edge_repr.T, nn2d, en2d)
    return _tc_finish(nsum, esum, node_repr, W_deg[:NODE_SIZE],
                      W_deg[NODE_SIZE:], W_self, bias)


# speedup vs baseline: 1.4841x; 1.0611x over previous
import functools

import jax
import jax.numpy as jnp
from jax import lax
from jax.experimental import pallas as pl
from jax.experimental.pallas import tpu as pltpu
from jax.experimental.pallas import tpu_sc as plsc

N_NODES = 10000
N_EDGES = 320000
DEGREE = 32
NODE_SIZE = 128
EDGE_SIZE = 16
OUT_SIZE = 128
EPS = 1e-5

NUM_WORKERS = 32
LANES = 16
IDX_ROW = 128
IDX_ROWS_TOTAL = N_NODES * DEGREE // IDX_ROW
IDX_PAD_ROWS = 2504
NBUF = 2

CHUNK_N = 8
MAX_CPW_N = -(-(N_NODES // CHUNK_N) // NUM_WORKERS)
CHUNK_E = 16
MAX_CPW_E = -(-(N_NODES // CHUNK_E) // NUM_WORKERS)

RP_BLK = 3200
PACKED_ROWS = N_EDGES * EDGE_SIZE // 128


def _worker_id():
    return lax.axis_index("s") * 2 + lax.axis_index("c")


def _tc_repack_body(in_ref, out_ref):
    x = in_ref[:]
    t = jnp.transpose(x)
    t3 = t.reshape(RP_BLK // 8, 8, EDGE_SIZE)
    out_ref[:] = jnp.concatenate([t3[:, s, :] for s in range(8)], axis=1)


def _repack_edges(edge_t):
    return pl.pallas_call(
        _tc_repack_body,
        grid=(N_EDGES // RP_BLK,),
        in_specs=[pl.BlockSpec((EDGE_SIZE, RP_BLK), lambda i: (0, i))],
        out_specs=pl.BlockSpec((RP_BLK // 8, 128), lambda i: (i, 0)),
        out_shape=jax.ShapeDtypeStruct((PACKED_ROWS, 128), jnp.float32),
    )(edge_t)


def _gather_sum_body(table_hbm, idx_hbm, out_hbm, idx_v, rows, out_v, sems,
                     *, width, chunk, max_cpw, aligned):
    nv = width // LANES
    rows_per_chunk = chunk * DEGREE // IDX_ROW
    wid = _worker_id()
    base = wid * (N_NODES // chunk) // NUM_WORKERS
    cnt = (wid + 1) * (N_NODES // chunk) // NUM_WORKERS - base
    irow0 = base * rows_per_chunk
    n_irows = max_cpw * rows_per_chunk

    if aligned:
        al = (irow0 // 8) * 8
        off = irow0 - al
        pltpu.sync_copy(idx_hbm.at[pl.ds(al, n_irows + 8)], idx_v)
    else:
        off = 0
        pltpu.sync_copy(idx_hbm.at[pl.ds(irow0, n_irows)], idx_v)

    def issue(k, b):
        for j in range(rows_per_chunk):
            pltpu.async_copy(
                table_hbm.at[idx_v.at[k * rows_per_chunk + j + off]],
                rows[b].at[pl.ds(j * IDX_ROW, IDX_ROW)], sems[b])

    def drain(k, b):
        for j in range(rows_per_chunk):
            pltpu.make_async_copy(
                table_hbm.at[idx_v.at[k * rows_per_chunk + j + off]],
                rows[b].at[pl.ds(j * IDX_ROW, IDX_ROW)], sems[b]).wait()

    def reduce_chunk(k, b):
        for n in range(chunk):
            def red(j, acc):
                new = acc
                for jj in range(4):
                    row = n * DEGREE + j * 4 + jj
                    new = tuple(
                        new[v] + rows[b][row, pl.ds(v * LANES, LANES)]
                        for v in range(nv)
                    )
                return new

            zero = jnp.zeros((LANES,), jnp.float32)
            acc = lax.fori_loop(0, DEGREE // 4, red, (zero,) * nv)
            out_row = k * chunk + n
            for v in range(nv):
                out_v[out_row, pl.ds(v * LANES, LANES)] = acc[v]

    for b in range(NBUF):
        issue(b, b)

    def pair_body(i, carry):
        for b in range(NBUF):
            k = i * NBUF + b

            @pl.when(k < cnt)
            def _():
                drain(k, b)
                reduce_chunk(k, b)

                @pl.when(k + NBUF < cnt)
                def _():
                    issue(k + NBUF, b)

        return carry

    lax.fori_loop(0, -(-max_cpw // NBUF), pair_body, 0)

    row0 = base * chunk

    @pl.when(cnt == max_cpw)
    def _():
        pltpu.sync_copy(out_v, out_hbm.at[pl.ds(row0, max_cpw * chunk)])

    @pl.when(cnt == max_cpw - 1)
    def _():
        small = (max_cpw - 1) * chunk
        pltpu.sync_copy(out_v.at[pl.ds(0, small)],
                        out_hbm.at[pl.ds(row0, small)])


def _make_sc_kernel(width, chunk, max_cpw, aligned, n_dummy=0):
    def body(*refs):
        _gather_sum_body(*refs[:2], *refs[2 + n_dummy:], width=width,
                         chunk=chunk, max_cpw=max_cpw, aligned=aligned)

    mesh = plsc.VectorSubcoreMesh(core_axis_name="c", subcore_axis_name="s")
    rows_per_chunk = chunk * DEGREE // IDX_ROW
    idx_rows = max_cpw * rows_per_chunk + (8 if aligned else 0)
    return pl.kernel(
        body,
        mesh=mesh,
        compiler_params=pltpu.CompilerParams(use_tc_tiling_on_sc=aligned),
        out_type=jax.ShapeDtypeStruct((N_NODES, width), jnp.float32),
        scratch_types=[
            pltpu.VMEM((idx_rows, IDX_ROW), jnp.int32),
            [pltpu.VMEM((rows_per_chunk * IDX_ROW, width), jnp.float32)
             for _ in range(NBUF)],
            pltpu.VMEM((max_cpw * chunk, width), jnp.float32),
            [pltpu.SemaphoreType.DMA for _ in range(NBUF)],
        ],
    )


@jax.jit
def _sc_gather_sums(node_repr, edge_t, nn2d, en2d):
    edge_packed = _repack_edges(edge_t)
    edge_lin = edge_packed.reshape(N_EDGES, EDGE_SIZE)
    nsum = _make_sc_kernel(NODE_SIZE, CHUNK_N, MAX_CPW_N, True)(
        node_repr, nn2d)
    esum = _make_sc_kernel(EDGE_SIZE, CHUNK_E, MAX_CPW_E, False, n_dummy=1)(
        edge_lin, en2d, nsum)
    return nsum, esum


def _tc_body(nsum_ref, esum_ref, node_ref, wdn_ref, wde_ref, ws_ref, bias_ref,
             out_ref):
    act = jnp.dot(nsum_ref[:], wdn_ref[:], preferred_element_type=jnp.float32)
    act = act + jnp.dot(esum_ref[:], wde_ref[:],
                        preferred_element_type=jnp.float32)
    act = act + jnp.dot(node_ref[:], ws_ref[:],
                        preferred_element_type=jnp.float32)
    act = act + bias_ref[:]
    mean = jnp.mean(act, axis=0, keepdims=True)
    cent = act - mean
    var = jnp.mean(cent * cent, axis=0, keepdims=True)
    out_ref[:] = jnp.maximum(cent * lax.rsqrt(var + EPS), 0.0)


def _tc_finish(nsum, esum, node_repr, wdn, wde, ws, bias):
    return pl.pallas_call(
        _tc_body,
        out_shape=jax.ShapeDtypeStruct((N_NODES, OUT_SIZE), jnp.float32),
    )(nsum, esum, node_repr, wdn, wde, ws, bias)


def kernel(node_repr, edge_repr, node_neighbor, edge_neighbor, W_deg, W_self,
           bias):
    nn2d = jnp.pad(node_neighbor.reshape(IDX_ROWS_TOTAL, IDX_ROW),
                   ((0, IDX_PAD_ROWS - IDX_ROWS_TOTAL), (0, 0)))
    en2d = edge_neighbor.reshape(IDX_ROWS_TOTAL, IDX_ROW)
    nsum, esum = _sc_gather_sums(node_repr, edge_repr.T, nn2d, en2d)
    return _tc_finish(nsum, esum, node_repr, W_deg[:NODE_SIZE],
                      W_deg[NODE_SIZE:], W_self, bias)
